# baseline (speedup 1.0000x reference)
import functools

import jax
import jax.numpy as jnp
from jax import lax
from jax.experimental import pallas as pl
from jax.experimental.pallas import tpu as pltpu
from jax.experimental.pallas import tpu_sc as plsc

DIM = 64
SUB = 128
GPC = 4
CH = SUB * GPC
NBUF = 2


def _gather_impl(x_flat, table):
    b_total = x_flat.shape[0]
    info = plsc.get_sparse_core_info()
    nw = info.num_cores * info.num_subcores
    rows_per_w = b_total // nw
    n_chunks = rows_per_w // CH
    mesh = plsc.VectorSubcoreMesh(core_axis_name="c", subcore_axis_name="s")

    @functools.partial(
        pl.kernel,
        mesh=mesh,
        compiler_params=pltpu.CompilerParams(use_tc_tiling_on_sc=False),
        out_type=jax.ShapeDtypeStruct((b_total, DIM), jnp.float32),
        scratch_types=[
            pltpu.VMEM((rows_per_w,), jnp.int32),
            pltpu.VMEM((NBUF, CH, DIM), jnp.float32),
            pltpu.SemaphoreType.DMA,
            pltpu.SemaphoreType.DMA,
        ],
    )
    def k(idx_hbm, table_hbm, out_hbm, idx_v, rows_v, gsem, osem):
        wid = lax.axis_index("s") * info.num_cores + lax.axis_index("c")
        base = wid * rows_per_w
        pltpu.sync_copy(idx_hbm.at[pl.ds(base, rows_per_w)], idx_v)

        def issue_gathers(g, slot):
            for j in range(GPC):
                pltpu.async_copy(
                    table_hbm.at[idx_v.at[pl.ds(g * CH + j * SUB, SUB)]],
                    rows_v.at[slot, pl.ds(j * SUB, SUB)],
                    gsem,
                )

        def drain(slot, sem):
            pltpu.make_async_copy(
                out_hbm.at[pl.ds(base, CH)], rows_v.at[slot], sem
            ).wait()

        def issue_out(g, slot):
            pltpu.async_copy(
                rows_v.at[slot],
                out_hbm.at[pl.ds(base + g * CH, CH)],
                osem,
            )

        issue_gathers(0, 0)

        def body(g, _):
            slot = lax.rem(g, NBUF)
            nxt = lax.rem(g + 1, NBUF)

            @pl.when(g + 1 < n_chunks)
            def _():
                @pl.when(g + 1 >= NBUF)
                def _():
                    drain(nxt, osem)

                issue_gathers(g + 1, nxt)

            drain(slot, gsem)
            issue_out(g, slot)
            return 0

        lax.fori_loop(0, n_chunks, body, 0)
        for t in range(min(NBUF, n_chunks)):
            drain((n_chunks - 1 - t) % NBUF, osem)

    return k(x_flat, table)


def kernel(x, table):
    b, s = x.shape
    out = _gather_impl(x.reshape(b * s), table)
    return out.reshape(b, s, DIM)
